# baseline (speedup 1.0000x reference)
import jax
import jax.numpy as jnp
from jax import lax
from jax.experimental import pallas as pl
from jax.experimental.pallas import tpu as pltpu
from jax.experimental.pallas import tpu_sc as plsc

B = 16384
XD = 512
YD = 64
NK = 64
HID = 2048
NP = NK // 2
TB = 256
NB = B // TB
TF = 1024
NF = B // TF

_NC = 2
_NS = 16
_NW = _NC * _NS
_BPW = B // _NW
_CH = 64
_NCH = _BPW // _CH


def _labels_body(y_ref, c_ref, lab_ref, toh_ref):
    y = y_ref[...]
    c = c_ref[...]
    d2 = (jnp.sum(y * y, axis=1, keepdims=True)
          - 2.0 * jax.lax.dot_general(y, c, (((1,), (1,)), ((), ())),
                                      preferred_element_type=jnp.float32)
          + jnp.sum(c * c, axis=1)[None, :])
    lab = jnp.argmin(d2, axis=1).astype(jnp.int32)
    lab_ref[...] = jnp.broadcast_to(lab[:, None], (TB, NK))
    lane = lax.broadcasted_iota(jnp.int32, (TB, NK), 1)
    oh = (lane == lab[:, None]).astype(jnp.float32)
    ct = jnp.dot(oh, c, preferred_element_type=jnp.float32)
    toh_ref[...] = jnp.concatenate([y - ct, oh], axis=1)


def _fused_body(plo_ref, pcnt_ref, xs_ref, w1_ref, b1_ref, w2_ref, b2_ref,
                w3l_ref, b3l_ref, w3r_ref, b3r_ref, toh_ref, acc_ref):
    i = pl.program_id(0)
    HF = TF // 2
    w1 = w1_ref[...]
    w2 = w2_ref[...]

    def mlp_half(sl):
        xs = xs_ref[sl, :].astype(jnp.bfloat16)
        h1 = jnp.dot(xs, w1, preferred_element_type=jnp.float32)
        h1 = jnp.maximum(h1 + b1_ref[...], 0.0).astype(jnp.bfloat16)
        h2 = jnp.dot(h1, w2, preferred_element_type=jnp.float32)
        return jnp.maximum(h2 + b2_ref[...], 0.0).astype(jnp.bfloat16)

    h2a = mlp_half(pl.ds(0, HF))
    h2b_ = mlp_half(pl.ds(HF, HF))
    h2 = jnp.concatenate([h2a, h2b_], axis=0)

    logits = jnp.dot(h2, w3l_ref[...], preferred_element_type=jnp.float32)
    logits = logits + b3l_ref[...]
    oh = toh_ref[:, YD:]
    t = toh_ref[:, :YD]
    lane = lax.broadcasted_iota(jnp.int32, (TF, NK), 1)
    plo = plo_ref[i]
    pcnt = pcnt_ref[i]

    def pair_term(q, valid):
        w = w3r_ref[:, pl.ds(q * 2 * YD, 2 * YD)]
        p = jnp.dot(h2, w, preferred_element_type=jnp.float32)
        p = p + b3r_ref[:, pl.ds(q * 2 * YD, 2 * YD)]
        sel_lo = jnp.where(valid, jnp.where(lane == 2 * q, oh, 0.0), 0.0)
        sel_hi = jnp.where(valid, jnp.where(lane == 2 * q + 1, oh, 0.0), 0.0)
        rs_lo = jnp.sum(sel_lo, axis=1, keepdims=True)
        rs_hi = jnp.sum(sel_hi, axis=1, keepdims=True)
        d_lo = t - p[:, :YD]
        d_hi = t - p[:, YD:]
        return (jnp.sum(d_lo * d_lo * rs_lo)
                + jnp.sum(d_hi * d_hi * rs_hi))

    mse_part = pair_term(plo, True)
    q1 = jnp.minimum(plo + 1, NP - 1)
    mse_part = mse_part + pair_term(q1, jnp.logical_and(pcnt > 1,
                                                        plo + 1 < NP))

    def pair_step(kk, acc):
        return acc + pair_term(plo + kk, True)

    mse_part = lax.fori_loop(2, pcnt, pair_step, mse_part)

    m = jnp.max(logits, axis=1, keepdims=True)
    lse = m[:, 0] + jnp.log(jnp.sum(jnp.exp(logits - m), axis=1))
    picked = jnp.sum(logits * oh, axis=1)
    ce_part = jnp.sum(lse - picked)

    part = jnp.concatenate([jnp.full((1, 1), ce_part, jnp.float32),
                            jnp.full((1, 1), mse_part, jnp.float32)], axis=1)
    prev = jnp.where(i == 0, jnp.zeros((1, 2), jnp.float32), acc_ref[...])
    acc_ref[...] = prev + part


def _sc_gather_body(x_hbm, toh_hbm, sidx_hbm,
                    g1_hbm, g2_hbm,
                    idx_v, b1a, b1b, b2a, b2b, sga, sgb, swa, swb):
    wid = lax.axis_index("s") * _NC + lax.axis_index("c")
    base = wid * _BPW
    pltpu.sync_copy(sidx_hbm.at[pl.ds(base, _BPW)], idx_v)

    bufs1 = (b1a, b1b)
    bufs2 = (b2a, b2b)
    gsems = (sga, sgb)
    wsems = (swa, swb)
    writes = [None, None]
    for ci in range(_NCH):
        bi = ci % 2
        if writes[bi] is not None:
            for w in writes[bi]:
                w.wait()
        idx_c = idx_v.at[pl.ds(ci * _CH, _CH)]
        c1 = pltpu.async_copy(x_hbm.at[idx_c], bufs1[bi], gsems[bi])
        c2 = pltpu.async_copy(toh_hbm.at[idx_c], bufs2[bi], gsems[bi])
        c1.wait()
        c2.wait()
        dst = pl.ds(base + ci * _CH, _CH)
        w1 = pltpu.async_copy(bufs1[bi], g1_hbm.at[dst], wsems[bi])
        w2 = pltpu.async_copy(bufs2[bi], g2_hbm.at[dst], wsems[bi])
        writes[bi] = (w1, w2)
    for ws in writes:
        if ws is not None:
            for w in ws:
                w.wait()


def _sc_gather(x_batch, toh, sidx):
    mesh = plsc.VectorSubcoreMesh(core_axis_name="c", subcore_axis_name="s",
                                  num_cores=_NC, num_subcores=_NS)
    f = pl.kernel(
        _sc_gather_body,
        out_type=[
            jax.ShapeDtypeStruct((B, XD), jnp.float32),
            jax.ShapeDtypeStruct((B, YD + NK), jnp.float32),
        ],
        mesh=mesh,
        scratch_types=[
            pltpu.VMEM((_BPW,), jnp.int32),
            pltpu.VMEM((_CH, XD), jnp.float32),
            pltpu.VMEM((_CH, XD), jnp.float32),
            pltpu.VMEM((_CH, YD + NK), jnp.float32),
            pltpu.VMEM((_CH, YD + NK), jnp.float32),
            pltpu.SemaphoreType.DMA,
            pltpu.SemaphoreType.DMA,
            pltpu.SemaphoreType.DMA,
            pltpu.SemaphoreType.DMA,
        ],
    )
    return f(x_batch, toh, sidx)


def _compute_labels(y_batch, centers):
    return pl.pallas_call(
        _labels_body,
        grid=(NB,),
        in_specs=[
            pl.BlockSpec((TB, YD), lambda i: (i, 0)),
            pl.BlockSpec((NK, YD), lambda i: (0, 0)),
        ],
        out_specs=[
            pl.BlockSpec((TB, NK), lambda i: (i, 0)),
            pl.BlockSpec((TB, YD + NK), lambda i: (i, 0)),
        ],
        out_shape=[
            jax.ShapeDtypeStruct((B, NK), jnp.int32),
            jax.ShapeDtypeStruct((B, YD + NK), jnp.float32),
        ],
    )(y_batch, centers)


def _fused_loss(plo, pcnt, g1, w1x, b1, w2, b2, w3l, b3l, w3rp, b3rp, g2):
    grid_spec = pltpu.PrefetchScalarGridSpec(
        num_scalar_prefetch=2,
        grid=(NF,),
        in_specs=[
            pl.BlockSpec((TF, XD), lambda i, plo, pcnt: (i, 0)),
            pl.BlockSpec((XD, HID), lambda i, plo, pcnt: (0, 0)),
            pl.BlockSpec((1, HID), lambda i, plo, pcnt: (0, 0)),
            pl.BlockSpec((HID, HID), lambda i, plo, pcnt: (0, 0)),
            pl.BlockSpec((1, HID), lambda i, plo, pcnt: (0, 0)),
            pl.BlockSpec((HID, NK), lambda i, plo, pcnt: (0, 0)),
            pl.BlockSpec((1, NK), lambda i, plo, pcnt: (0, 0)),
            pl.BlockSpec((HID, NP * 2 * YD), lambda i, plo, pcnt: (0, 0)),
            pl.BlockSpec((1, NP * 2 * YD), lambda i, plo, pcnt: (0, 0)),
            pl.BlockSpec((TF, YD + NK), lambda i, plo, pcnt: (i, 0)),
        ],
        out_specs=pl.BlockSpec((1, 2), lambda i, plo, pcnt: (0, 0)),
    )
    return pl.pallas_call(
        _fused_body,
        grid_spec=grid_spec,
        out_shape=jax.ShapeDtypeStruct((1, 2), jnp.float32),
    )(plo, pcnt, g1, w1x, b1, w2, b2, w3l, b3l, w3rp, b3rp, g2)


def kernel(x_batch, y_batch, W1, b1, W2, b2, W3, b3, centers):
    lab_full, toh = _compute_labels(y_batch, centers)
    labels = lab_full[:, 0]

    packed = jnp.sort((labels << 14) | jnp.arange(B, dtype=jnp.int32))
    sidx = packed & (B - 1)
    slab = packed >> 14
    blk = slab.reshape(NF, TF)
    plo = (blk[:, 0] // 2).astype(jnp.int32)
    pcnt = (blk[:, -1] // 2 - blk[:, 0] // 2 + 1).astype(jnp.int32)

    g1, g2 = _sc_gather(x_batch, toh, sidx)

    w1x = W1[YD:YD + XD, :].astype(jnp.bfloat16)
    w2 = W2.astype(jnp.bfloat16)
    w3l = W3[:, :NK].astype(jnp.bfloat16)
    w3rp = W3[:, NK:].astype(jnp.bfloat16)
    b3rp = b3[NK:].reshape(1, NP * 2 * YD)
    acc = _fused_loss(plo, pcnt, g1, w1x, b1.reshape(1, HID), w2,
                      b2.reshape(1, HID), w3l, b3[:NK].reshape(1, NK),
                      w3rp, b3rp, g2)

    return acc[0, 0] / B + 100.0 * acc[0, 1] / (B * YD)


# speedup vs baseline: 1.0028x; 1.0028x over previous
import jax
import jax.numpy as jnp
from jax import lax
from jax.experimental import pallas as pl
from jax.experimental.pallas import tpu as pltpu
from jax.experimental.pallas import tpu_sc as plsc

B = 16384
XD = 512
YD = 64
NK = 64
HID = 2048
NP = NK // 2
TB = 256
NB = B // TB
TF = 512
NF = B // TF

_NC = 2
_NS = 16
_NW = _NC * _NS
_BPW = B // _NW
_CH = 64
_NCH = _BPW // _CH


def _labels_body(y_ref, c_ref, lab_ref, toh_ref):
    y = y_ref[...]
    c = c_ref[...]
    d2 = (jnp.sum(y * y, axis=1, keepdims=True)
          - 2.0 * jax.lax.dot_general(y, c, (((1,), (1,)), ((), ())),
                                      preferred_element_type=jnp.float32)
          + jnp.sum(c * c, axis=1)[None, :])
    lab = jnp.argmin(d2, axis=1).astype(jnp.int32)
    lab_ref[...] = jnp.broadcast_to(lab[:, None], (TB, NK))
    lane = lax.broadcasted_iota(jnp.int32, (TB, NK), 1)
    oh = (lane == lab[:, None]).astype(jnp.float32)
    ct = jnp.dot(oh, c, preferred_element_type=jnp.float32)
    toh_ref[...] = jnp.concatenate([y - ct, oh], axis=1)


def _fused_body(plo_ref, pcnt_ref, xs_ref, w1_ref, b1_ref, w2_ref, b2_ref,
                w3l_ref, b3l_ref, w3r_ref, b3r_ref, toh_ref, acc_ref):
    i = pl.program_id(0)
    HF = TF // 2
    w1 = w1_ref[...]
    w2 = w2_ref[...]

    def mlp_half(sl):
        xs = xs_ref[sl, :].astype(jnp.bfloat16)
        h1 = jnp.dot(xs, w1, preferred_element_type=jnp.float32)
        h1 = jnp.maximum(h1 + b1_ref[...], 0.0).astype(jnp.bfloat16)
        h2 = jnp.dot(h1, w2, preferred_element_type=jnp.float32)
        return jnp.maximum(h2 + b2_ref[...], 0.0).astype(jnp.bfloat16)

    h2a = mlp_half(pl.ds(0, HF))
    h2b_ = mlp_half(pl.ds(HF, HF))
    h2 = jnp.concatenate([h2a, h2b_], axis=0)

    logits = jnp.dot(h2, w3l_ref[...], preferred_element_type=jnp.float32)
    logits = logits + b3l_ref[...]
    oh = toh_ref[:, YD:]
    t = toh_ref[:, :YD]
    lane = lax.broadcasted_iota(jnp.int32, (TF, NK), 1)
    plo = plo_ref[i]
    pcnt = pcnt_ref[i]

    def pair_term(q, valid):
        w = w3r_ref[:, pl.ds(q * 2 * YD, 2 * YD)]
        p = jnp.dot(h2, w, preferred_element_type=jnp.float32)
        p = p + b3r_ref[:, pl.ds(q * 2 * YD, 2 * YD)]
        sel_lo = jnp.where(valid, jnp.where(lane == 2 * q, oh, 0.0), 0.0)
        sel_hi = jnp.where(valid, jnp.where(lane == 2 * q + 1, oh, 0.0), 0.0)
        rs_lo = jnp.sum(sel_lo, axis=1, keepdims=True)
        rs_hi = jnp.sum(sel_hi, axis=1, keepdims=True)
        d_lo = t - p[:, :YD]
        d_hi = t - p[:, YD:]
        return (jnp.sum(d_lo * d_lo * rs_lo)
                + jnp.sum(d_hi * d_hi * rs_hi))

    mse_part = pair_term(plo, True)
    q1 = jnp.minimum(plo + 1, NP - 1)
    mse_part = mse_part + pair_term(q1, jnp.logical_and(pcnt > 1,
                                                        plo + 1 < NP))

    def pair_step(kk, acc):
        return acc + pair_term(plo + kk, True)

    mse_part = lax.fori_loop(2, pcnt, pair_step, mse_part)

    m = jnp.max(logits, axis=1, keepdims=True)
    lse = m[:, 0] + jnp.log(jnp.sum(jnp.exp(logits - m), axis=1))
    picked = jnp.sum(logits * oh, axis=1)
    ce_part = jnp.sum(lse - picked)

    part = jnp.concatenate([jnp.full((1, 1), ce_part, jnp.float32),
                            jnp.full((1, 1), mse_part, jnp.float32)], axis=1)
    prev = jnp.where(i == 0, jnp.zeros((1, 2), jnp.float32), acc_ref[...])
    acc_ref[...] = prev + part


def _sc_gather_body(x_hbm, toh_hbm, sidx_hbm,
                    g1_hbm, g2_hbm,
                    idx_v, b1a, b1b, b2a, b2b, sga, sgb, swa, swb):
    wid = lax.axis_index("s") * _NC + lax.axis_index("c")
    base = wid * _BPW
    pltpu.sync_copy(sidx_hbm.at[pl.ds(base, _BPW)], idx_v)

    bufs1 = (b1a, b1b)
    bufs2 = (b2a, b2b)
    gsems = (sga, sgb)
    wsems = (swa, swb)
    writes = [None, None]
    for ci in range(_NCH):
        bi = ci % 2
        if writes[bi] is not None:
            for w in writes[bi]:
                w.wait()
        idx_c = idx_v.at[pl.ds(ci * _CH, _CH)]
        c1 = pltpu.async_copy(x_hbm.at[idx_c], bufs1[bi], gsems[bi])
        c2 = pltpu.async_copy(toh_hbm.at[idx_c], bufs2[bi], gsems[bi])
        c1.wait()
        c2.wait()
        dst = pl.ds(base + ci * _CH, _CH)
        w1 = pltpu.async_copy(bufs1[bi], g1_hbm.at[dst], wsems[bi])
        w2 = pltpu.async_copy(bufs2[bi], g2_hbm.at[dst], wsems[bi])
        writes[bi] = (w1, w2)
    for ws in writes:
        if ws is not None:
            for w in ws:
                w.wait()


def _sc_gather(x_batch, toh, sidx):
    mesh = plsc.VectorSubcoreMesh(core_axis_name="c", subcore_axis_name="s",
                                  num_cores=_NC, num_subcores=_NS)
    f = pl.kernel(
        _sc_gather_body,
        out_type=[
            jax.ShapeDtypeStruct((B, XD), jnp.float32),
            jax.ShapeDtypeStruct((B, YD + NK), jnp.float32),
        ],
        mesh=mesh,
        scratch_types=[
            pltpu.VMEM((_BPW,), jnp.int32),
            pltpu.VMEM((_CH, XD), jnp.float32),
            pltpu.VMEM((_CH, XD), jnp.float32),
            pltpu.VMEM((_CH, YD + NK), jnp.float32),
            pltpu.VMEM((_CH, YD + NK), jnp.float32),
            pltpu.SemaphoreType.DMA,
            pltpu.SemaphoreType.DMA,
            pltpu.SemaphoreType.DMA,
            pltpu.SemaphoreType.DMA,
        ],
    )
    return f(x_batch, toh, sidx)


def _compute_labels(y_batch, centers):
    return pl.pallas_call(
        _labels_body,
        grid=(NB,),
        in_specs=[
            pl.BlockSpec((TB, YD), lambda i: (i, 0)),
            pl.BlockSpec((NK, YD), lambda i: (0, 0)),
        ],
        out_specs=[
            pl.BlockSpec((TB, NK), lambda i: (i, 0)),
            pl.BlockSpec((TB, YD + NK), lambda i: (i, 0)),
        ],
        out_shape=[
            jax.ShapeDtypeStruct((B, NK), jnp.int32),
            jax.ShapeDtypeStruct((B, YD + NK), jnp.float32),
        ],
    )(y_batch, centers)


def _fused_loss(plo, pcnt, g1, w1x, b1, w2, b2, w3l, b3l, w3rp, b3rp, g2):
    grid_spec = pltpu.PrefetchScalarGridSpec(
        num_scalar_prefetch=2,
        grid=(NF,),
        in_specs=[
            pl.BlockSpec((TF, XD), lambda i, plo, pcnt: (i, 0)),
            pl.BlockSpec((XD, HID), lambda i, plo, pcnt: (0, 0)),
            pl.BlockSpec((1, HID), lambda i, plo, pcnt: (0, 0)),
            pl.BlockSpec((HID, HID), lambda i, plo, pcnt: (0, 0)),
            pl.BlockSpec((1, HID), lambda i, plo, pcnt: (0, 0)),
            pl.BlockSpec((HID, NK), lambda i, plo, pcnt: (0, 0)),
            pl.BlockSpec((1, NK), lambda i, plo, pcnt: (0, 0)),
            pl.BlockSpec((HID, NP * 2 * YD), lambda i, plo, pcnt: (0, 0)),
            pl.BlockSpec((1, NP * 2 * YD), lambda i, plo, pcnt: (0, 0)),
            pl.BlockSpec((TF, YD + NK), lambda i, plo, pcnt: (i, 0)),
        ],
        out_specs=pl.BlockSpec((1, 2), lambda i, plo, pcnt: (0, 0)),
    )
    return pl.pallas_call(
        _fused_body,
        grid_spec=grid_spec,
        out_shape=jax.ShapeDtypeStruct((1, 2), jnp.float32),
    )(plo, pcnt, g1, w1x, b1, w2, b2, w3l, b3l, w3rp, b3rp, g2)


def kernel(x_batch, y_batch, W1, b1, W2, b2, W3, b3, centers):
    lab_full, toh = _compute_labels(y_batch, centers)
    labels = lab_full[:, 0]

    packed = jnp.sort((labels << 14) | jnp.arange(B, dtype=jnp.int32))
    sidx = packed & (B - 1)
    slab = packed >> 14
    blk = slab.reshape(NF, TF)
    plo = (blk[:, 0] // 2).astype(jnp.int32)
    pcnt = (blk[:, -1] // 2 - blk[:, 0] // 2 + 1).astype(jnp.int32)

    g1, g2 = _sc_gather(x_batch, toh, sidx)

    w1x = W1[YD:YD + XD, :].astype(jnp.bfloat16)
    w2 = W2.astype(jnp.bfloat16)
    w3l = W3[:, :NK].astype(jnp.bfloat16)
    w3rp = W3[:, NK:].astype(jnp.bfloat16)
    b3rp = b3[NK:].reshape(1, NP * 2 * YD)
    acc = _fused_loss(plo, pcnt, g1, w1x, b1.reshape(1, HID), w2,
                      b2.reshape(1, HID), w3l, b3[:NK].reshape(1, NK),
                      w3rp, b3rp, g2)

    return acc[0, 0] / B + 100.0 * acc[0, 1] / (B * YD)
